# baseline (speedup 1.0000x reference)
import math

import jax
import jax.numpy as jnp
from jax.experimental import pallas as pl
from jax.experimental.pallas import tpu as pltpu

_DIM = 1024
_H = 16
_HD = 64
_S = 2048
_K = 64
_RB = 256


def _gemm_bias_kernel(x_ref, w_ref, b_ref, o_ref):
    o_ref[...] = jax.lax.dot_general(
        x_ref[...], w_ref[...], (((1,), (0,)), ((), ())),
        preferred_element_type=jnp.float32) + b_ref[...]


def _attn_kernel(q_ref, k_ref, v_ref, routes_ref, wp_ref, bp_ref, o_ref, acc_ref):
    scale = 1.0 / math.sqrt(_HD)
    ids = jax.lax.broadcasted_iota(jnp.int32, (_RB, _S), 1)
    r = jnp.clip(routes_ref[...], 0, _S - 1)
    hit = jnp.zeros((_RB, _S), jnp.bool_)
    for j in range(_K):
        hit = jnp.logical_or(hit, ids == r[:, j][:, None])
    bias = jnp.where(hit, 0.0, -1e30).astype(jnp.float32)
    for h in range(_H):
        sl = slice(h * _HD, (h + 1) * _HD)
        s = jax.lax.dot_general(
            q_ref[:, sl], k_ref[:, sl], (((1,), (1,)), ((), ())),
            preferred_element_type=jnp.float32)
        s = s * scale + bias
        m = jnp.max(s, axis=-1, keepdims=True)
        e = jnp.exp(s - m)
        p = e / jnp.sum(e, axis=-1, keepdims=True)
        acc_ref[:, sl] = jax.lax.dot_general(
            p, v_ref[:, sl], (((1,), (0,)), ((), ())),
            preferred_element_type=jnp.float32)
    o_ref[...] = jax.lax.dot_general(
        acc_ref[...], wp_ref[...], (((1,), (0,)), ((), ())),
        preferred_element_type=jnp.float32) + bp_ref[...]


def kernel(x, W_qkv, b_qkv, W_proj, b_proj, routes):
    B, S, D = x.shape
    x2 = x.reshape(S, D)
    b_qkv2 = b_qkv.reshape(1, 3 * D)
    b_proj2 = b_proj.reshape(1, D)
    routes = routes.astype(jnp.int32)

    nrb = S // _RB

    qkv = pl.pallas_call(
        _gemm_bias_kernel,
        grid=(nrb, 3),
        in_specs=[
            pl.BlockSpec((_RB, D), lambda i, j: (i, 0)),
            pl.BlockSpec((D, D), lambda i, j: (0, j)),
            pl.BlockSpec((1, D), lambda i, j: (0, j)),
        ],
        out_specs=pl.BlockSpec((_RB, D), lambda i, j: (i, j)),
        out_shape=jax.ShapeDtypeStruct((S, 3 * D), jnp.float32),
    )(x2, W_qkv, b_qkv2)

    out = pl.pallas_call(
        _attn_kernel,
        grid=(nrb,),
        in_specs=[
            pl.BlockSpec((_RB, D), lambda i: (i, 0)),
            pl.BlockSpec((S, D), lambda i: (0, 1)),
            pl.BlockSpec((S, D), lambda i: (0, 2)),
            pl.BlockSpec((_RB, _K), lambda i: (i, 0)),
            pl.BlockSpec((D, D), lambda i: (0, 0)),
            pl.BlockSpec((1, D), lambda i: (0, 0)),
        ],
        out_specs=pl.BlockSpec((_RB, D), lambda i: (i, 0)),
        out_shape=jax.ShapeDtypeStruct((S, D), jnp.float32),
        scratch_shapes=[pltpu.VMEM((_RB, D), jnp.float32)],
    )(qkv, qkv, qkv, routes, W_proj, b_proj2)

    return out.reshape(B, S, D)


# speedup vs baseline: 1.2579x; 1.2579x over previous
import math

import jax
from jax import lax
import jax.numpy as jnp
from jax.experimental import pallas as pl
from jax.experimental.pallas import tpu as pltpu

_DIM = 1024
_H = 16
_HD = 64
_S = 2048
_K = 64
_RB = 256


def _gemm_bias_kernel(x_ref, w_ref, b_ref, o_ref):
    o_ref[...] = jax.lax.dot_general(
        x_ref[...], w_ref[...], (((1,), (0,)), ((), ())),
        preferred_element_type=jnp.float32) + b_ref[...]


def _bias_kernel(routes_ref, o_ref):
    iota = lax.broadcasted_iota(jnp.int32, (_RB, _S), 1)
    m = iota == routes_ref[:, 0:1]
    for k in range(1, _K):
        m = m | (iota == routes_ref[:, k:k + 1])
    o_ref[...] = jnp.where(m, 0.0, -1e30)


def _attn_kernel(q_ref, k_ref, v_ref, bias_ref, wp_ref, bp_ref, o_ref, acc_ref):
    scale = 1.0 / math.sqrt(_HD)
    bias = bias_ref[...]
    for h in range(_H):
        sl = slice(h * _HD, (h + 1) * _HD)
        s = jax.lax.dot_general(
            q_ref[:, sl], k_ref[:, sl], (((1,), (1,)), ((), ())),
            preferred_element_type=jnp.float32)
        s = s * scale + bias
        m = jnp.max(s, axis=-1, keepdims=True)
        e = jnp.exp(s - m)
        p = e / jnp.sum(e, axis=-1, keepdims=True)
        acc_ref[:, sl] = jax.lax.dot_general(
            p, v_ref[:, sl], (((1,), (0,)), ((), ())),
            preferred_element_type=jnp.float32)
    o_ref[...] = jax.lax.dot_general(
        acc_ref[...], wp_ref[...], (((1,), (0,)), ((), ())),
        preferred_element_type=jnp.float32) + bp_ref[...]


def kernel(x, W_qkv, b_qkv, W_proj, b_proj, routes):
    B, S, D = x.shape
    x2 = x.reshape(S, D)
    b_qkv2 = b_qkv.reshape(1, 3 * D)
    b_proj2 = b_proj.reshape(1, D)
    routes = routes.astype(jnp.int32)

    nrb = S // _RB

    qkv = pl.pallas_call(
        _gemm_bias_kernel,
        grid=(nrb, 3),
        in_specs=[
            pl.BlockSpec((_RB, D), lambda i, j: (i, 0)),
            pl.BlockSpec((D, D), lambda i, j: (0, j)),
            pl.BlockSpec((1, D), lambda i, j: (0, j)),
        ],
        out_specs=pl.BlockSpec((_RB, D), lambda i, j: (i, j)),
        out_shape=jax.ShapeDtypeStruct((S, 3 * D), jnp.float32),
    )(x2, W_qkv, b_qkv2)

    bias = pl.pallas_call(
        _bias_kernel,
        grid=(nrb,),
        in_specs=[pl.BlockSpec((_RB, _K), lambda i: (i, 0))],
        out_specs=pl.BlockSpec((_RB, _S), lambda i: (i, 0)),
        out_shape=jax.ShapeDtypeStruct((S, _S), jnp.float32),
    )(routes.reshape(S, _K))

    out = pl.pallas_call(
        _attn_kernel,
        grid=(nrb,),
        in_specs=[
            pl.BlockSpec((_RB, D), lambda i: (i, 0)),
            pl.BlockSpec((S, D), lambda i: (0, 1)),
            pl.BlockSpec((S, D), lambda i: (0, 2)),
            pl.BlockSpec((_RB, _S), lambda i: (i, 0)),
            pl.BlockSpec((D, D), lambda i: (0, 0)),
            pl.BlockSpec((1, D), lambda i: (0, 0)),
        ],
        out_specs=pl.BlockSpec((_RB, D), lambda i: (i, 0)),
        out_shape=jax.ShapeDtypeStruct((S, D), jnp.float32),
        scratch_shapes=[pltpu.VMEM((_RB, D), jnp.float32)],
    )(qkv, qkv, qkv, bias, W_proj, b_proj2)

    return out.reshape(B, S, D)


# speedup vs baseline: 1.2866x; 1.0228x over previous
import math

import jax
from jax import lax
import jax.numpy as jnp
from jax.experimental import pallas as pl
from jax.experimental.pallas import tpu as pltpu

_DIM = 1024
_H = 16
_HD = 64
_S = 2048
_K = 64
_RB = 256


def _gemm_bias_kernel(x_ref, w_ref, b_ref, o_ref):
    o_ref[...] = jax.lax.dot_general(
        x_ref[...], w_ref[...], (((1,), (0,)), ((), ())),
        preferred_element_type=jnp.float32) + b_ref[...]


def _attn_kernel(routes_ref, q_ref, k_ref, v_ref, wp_ref, bp_ref, o_ref,
                 acc_ref):
    iota = lax.broadcasted_iota(jnp.int32, (_RB, _S), 1)
    msk = iota == routes_ref[:, 0:1]
    for kk in range(1, _K):
        msk = msk | (iota == routes_ref[:, kk:kk + 1])
    bias = jnp.where(msk, 0.0, -1e30)

    scale = 1.0 / math.sqrt(_HD)
    for h in range(_H):
        sl = slice(h * _HD, (h + 1) * _HD)
        s = jax.lax.dot_general(
            q_ref[:, sl], k_ref[:, sl], (((1,), (1,)), ((), ())),
            preferred_element_type=jnp.float32)
        s = s * scale + bias
        m = jnp.max(s, axis=-1, keepdims=True)
        e = jnp.exp(s - m)
        u = jax.lax.dot_general(
            e, v_ref[:, sl], (((1,), (0,)), ((), ())),
            preferred_element_type=jnp.float32)
        acc_ref[:, sl] = u / jnp.sum(e, axis=-1, keepdims=True)
    o_ref[...] = jax.lax.dot_general(
        acc_ref[...], wp_ref[...], (((1,), (0,)), ((), ())),
        preferred_element_type=jnp.float32) + bp_ref[...]


def kernel(x, W_qkv, b_qkv, W_proj, b_proj, routes):
    B, S, D = x.shape
    x2 = x.reshape(S, D)
    b_qkv2 = b_qkv.reshape(1, 3 * D)
    b_proj2 = b_proj.reshape(1, D)
    routes = routes.astype(jnp.int32)

    nrb = S // _RB

    qkv = pl.pallas_call(
        _gemm_bias_kernel,
        grid=(nrb,),
        in_specs=[
            pl.BlockSpec((_RB, D), lambda i: (i, 0)),
            pl.BlockSpec((D, 3 * D), lambda i: (0, 0)),
            pl.BlockSpec((1, 3 * D), lambda i: (0, 0)),
        ],
        out_specs=pl.BlockSpec((_RB, 3 * D), lambda i: (i, 0)),
        out_shape=jax.ShapeDtypeStruct((S, 3 * D), jnp.float32),
    )(x2, W_qkv, b_qkv2)

    out = pl.pallas_call(
        _attn_kernel,
        grid=(nrb,),
        in_specs=[
            pl.BlockSpec((_RB, _K), lambda i: (i, 0)),
            pl.BlockSpec((_RB, D), lambda i: (i, 0)),
            pl.BlockSpec((S, D), lambda i: (0, 1)),
            pl.BlockSpec((S, D), lambda i: (0, 2)),
            pl.BlockSpec((D, D), lambda i: (0, 0)),
            pl.BlockSpec((1, D), lambda i: (0, 0)),
        ],
        out_specs=pl.BlockSpec((_RB, D), lambda i: (i, 0)),
        out_shape=jax.ShapeDtypeStruct((S, D), jnp.float32),
        scratch_shapes=[pltpu.VMEM((_RB, D), jnp.float32)],
    )(routes.reshape(S, _K), qkv, qkv, qkv, W_proj, b_proj2)

    return out.reshape(B, S, D)


# speedup vs baseline: 1.3190x; 1.0251x over previous
import math

import jax
from jax import lax
import jax.numpy as jnp
from jax.experimental import pallas as pl
from jax.experimental.pallas import tpu as pltpu

_DIM = 1024
_H = 16
_HD = 64
_S = 2048
_K = 64
_RB = 256


def _gemm_bias_kernel(x_ref, w_ref, b_ref, o_ref):
    acc = jax.lax.dot_general(
        x_ref[...], w_ref[...], (((1,), (0,)), ((), ())),
        preferred_element_type=jnp.float32) + b_ref[...]
    o_ref[...] = acc.astype(jnp.bfloat16)


def _attn_kernel(routes_ref, q_ref, k_ref, v_ref, wp_ref, bp_ref, o_ref):
    iota = lax.broadcasted_iota(jnp.int32, (_RB, _S), 1)
    msk = iota == routes_ref[:, 0:1]
    for kk in range(1, _K):
        msk = msk | (iota == routes_ref[:, kk:kk + 1])
    bias = jnp.where(msk, 0.0, -1e30)

    scale = 1.0 / math.sqrt(_HD)
    acc = []
    for h in range(_H):
        sl = slice(h * _HD, (h + 1) * _HD)
        s = jax.lax.dot_general(
            q_ref[:, sl], k_ref[:, sl], (((1,), (1,)), ((), ())),
            preferred_element_type=jnp.float32)
        s = s * scale + bias
        m = jnp.max(s, axis=-1, keepdims=True)
        e = jnp.exp(s - m)
        u = jax.lax.dot_general(
            e.astype(jnp.bfloat16), v_ref[:, sl], (((1,), (0,)), ((), ())),
            preferred_element_type=jnp.float32)
        acc.append(u / jnp.sum(e, axis=-1, keepdims=True))
    accm = jnp.concatenate(acc, axis=1).astype(jnp.bfloat16)
    o_ref[...] = jax.lax.dot_general(
        accm, wp_ref[...], (((1,), (0,)), ((), ())),
        preferred_element_type=jnp.float32) + bp_ref[...]


def kernel(x, W_qkv, b_qkv, W_proj, b_proj, routes):
    B, S, D = x.shape
    x2 = x.reshape(S, D).astype(jnp.bfloat16)
    W_qkv16 = W_qkv.astype(jnp.bfloat16)
    W_proj16 = W_proj.astype(jnp.bfloat16)
    b_qkv2 = b_qkv.reshape(1, 3 * D)
    b_proj2 = b_proj.reshape(1, D)
    routes = routes.astype(jnp.int32)

    nrb = S // _RB

    qkv = pl.pallas_call(
        _gemm_bias_kernel,
        grid=(nrb,),
        in_specs=[
            pl.BlockSpec((_RB, D), lambda i: (i, 0)),
            pl.BlockSpec((D, 3 * D), lambda i: (0, 0)),
            pl.BlockSpec((1, 3 * D), lambda i: (0, 0)),
        ],
        out_specs=pl.BlockSpec((_RB, 3 * D), lambda i: (i, 0)),
        out_shape=jax.ShapeDtypeStruct((S, 3 * D), jnp.bfloat16),
    )(x2, W_qkv16, b_qkv2)

    out = pl.pallas_call(
        _attn_kernel,
        grid=(nrb,),
        in_specs=[
            pl.BlockSpec((_RB, _K), lambda i: (i, 0)),
            pl.BlockSpec((_RB, D), lambda i: (i, 0)),
            pl.BlockSpec((S, D), lambda i: (0, 1)),
            pl.BlockSpec((S, D), lambda i: (0, 2)),
            pl.BlockSpec((D, D), lambda i: (0, 0)),
            pl.BlockSpec((1, D), lambda i: (0, 0)),
        ],
        out_specs=pl.BlockSpec((_RB, D), lambda i: (i, 0)),
        out_shape=jax.ShapeDtypeStruct((S, D), jnp.float32),
    )(routes.reshape(S, _K), qkv, qkv, qkv, W_proj16, b_proj2)

    return out.reshape(B, S, D)


# speedup vs baseline: 2.0231x; 1.5339x over previous
import math

import jax
from jax import lax
import jax.numpy as jnp
from jax.experimental import pallas as pl
from jax.experimental.pallas import tpu as pltpu

_DIM = 1024
_H = 16
_HD = 64
_S = 2048
_K = 64
_RB = 256


def _gemm_bias_kernel(x_ref, w_ref, b_ref, o_ref):
    acc = jax.lax.dot_general(
        x_ref[...], w_ref[...], (((1,), (0,)), ((), ())),
        preferred_element_type=jnp.float32) + b_ref[...]
    o_ref[...] = acc.astype(jnp.bfloat16)


def _attn_kernel(routes_ref, q_ref, k_ref, v_ref, wp_ref, bp_ref, o_ref):
    routes = routes_ref[...]
    rj = routes & 63
    rbit = jnp.int32(1) << (routes >> 6)
    jiota = lax.broadcasted_iota(jnp.int32, (_RB, _K), 1)
    bits = jnp.zeros((_RB, _K), jnp.int32)
    for kk in range(_K):
        bits = bits | jnp.where(rj[:, kk:kk + 1] == jiota,
                                rbit[:, kk:kk + 1], 0)
    words = jnp.tile(bits, (1, _S // _K))
    iota = lax.broadcasted_iota(jnp.int32, (_RB, _S), 1)
    colbit = jnp.int32(1) << (iota >> 6)
    bias = jnp.where((words & colbit) != 0, 0.0, -1e30)

    scale = 1.0 / math.sqrt(_HD)
    acc = []
    for h in range(_H):
        sl = slice(h * _HD, (h + 1) * _HD)
        s = jax.lax.dot_general(
            q_ref[:, sl], k_ref[:, sl], (((1,), (1,)), ((), ())),
            preferred_element_type=jnp.float32)
        s = s * scale + bias
        m = jnp.max(s, axis=-1, keepdims=True)
        e = jnp.exp(s - m)
        u = jax.lax.dot_general(
            e.astype(jnp.bfloat16), v_ref[:, sl], (((1,), (0,)), ((), ())),
            preferred_element_type=jnp.float32)
        acc.append(u / jnp.sum(e, axis=-1, keepdims=True))
    accm = jnp.concatenate(acc, axis=1).astype(jnp.bfloat16)
    o_ref[...] = jax.lax.dot_general(
        accm, wp_ref[...], (((1,), (0,)), ((), ())),
        preferred_element_type=jnp.float32) + bp_ref[...]


def kernel(x, W_qkv, b_qkv, W_proj, b_proj, routes):
    B, S, D = x.shape
    x2 = x.reshape(S, D).astype(jnp.bfloat16)
    W_qkv16 = W_qkv.astype(jnp.bfloat16)
    W_proj16 = W_proj.astype(jnp.bfloat16)
    b_qkv2 = b_qkv.reshape(1, 3 * D)
    b_proj2 = b_proj.reshape(1, D)
    routes = routes.astype(jnp.int32)

    nrb = S // _RB

    qkv = pl.pallas_call(
        _gemm_bias_kernel,
        grid=(nrb,),
        in_specs=[
            pl.BlockSpec((_RB, D), lambda i: (i, 0)),
            pl.BlockSpec((D, 3 * D), lambda i: (0, 0)),
            pl.BlockSpec((1, 3 * D), lambda i: (0, 0)),
        ],
        out_specs=pl.BlockSpec((_RB, 3 * D), lambda i: (i, 0)),
        out_shape=jax.ShapeDtypeStruct((S, 3 * D), jnp.bfloat16),
    )(x2, W_qkv16, b_qkv2)

    out = pl.pallas_call(
        _attn_kernel,
        grid=(nrb,),
        in_specs=[
            pl.BlockSpec((_RB, _K), lambda i: (i, 0)),
            pl.BlockSpec((_RB, D), lambda i: (i, 0)),
            pl.BlockSpec((S, D), lambda i: (0, 1)),
            pl.BlockSpec((S, D), lambda i: (0, 2)),
            pl.BlockSpec((D, D), lambda i: (0, 0)),
            pl.BlockSpec((1, D), lambda i: (0, 0)),
        ],
        out_specs=pl.BlockSpec((_RB, D), lambda i: (i, 0)),
        out_shape=jax.ShapeDtypeStruct((S, D), jnp.float32),
    )(routes.reshape(S, _K), qkv, qkv, qkv, W_proj16, b_proj2)

    return out.reshape(B, S, D)
